# baseline (speedup 1.0000x reference)
import functools

import jax
import jax.numpy as jnp
from jax import lax
from jax.experimental import pallas as pl
from jax.experimental.pallas import tpu as pltpu
from jax.experimental.pallas import tpu_sc as plsc

NC = 2
NS = 16
NW = NC * NS
L = 16

N = 10000
E = 320000
N_PAD = 10112
ROWS_PER_TILE = N_PAD // NS
ECH = 1000
EPT = E // NW
NCH = EPT // ECH

_mesh = plsc.VectorSubcoreMesh(core_axis_name="c", subcore_axis_name="s")
_sc_params = pltpu.CompilerParams(use_tc_tiling_on_sc=False)


@functools.partial(
    pl.kernel,
    mesh=_mesh,
    out_type=jax.ShapeDtypeStruct((NC, N_PAD, L), jnp.float32),
    scratch_types=[
        pltpu.VMEM((NCH, ECH), jnp.int32),
        pltpu.VMEM((ECH, L), jnp.float32),
        pltpu.VMEM_SHARED((N_PAD, L), jnp.float32),
    ],
    compiler_params=_sc_params,
)
def _sc_deg(dst_hbm, ones_hbm, zeros_hbm, out_hbm, dst_v, ones_v, deg_sh):
    cid = lax.axis_index("c")
    sid = lax.axis_index("s")
    wid = sid * NC + cid
    row0 = sid * ROWS_PER_TILE
    pltpu.sync_copy(zeros_hbm, deg_sh.at[pl.ds(row0, ROWS_PER_TILE)])
    pltpu.sync_copy(ones_hbm, ones_v)
    for j in range(NCH):
        pltpu.sync_copy(dst_hbm.at[pl.ds(wid * EPT + j * ECH, ECH)], dst_v.at[j])
    plsc.subcore_barrier()

    def step(j, c):
        pltpu.sync_copy(ones_v, deg_sh.at[dst_v.at[j]], add=True)
        return c

    lax.fori_loop(0, NCH, step, 0)
    plsc.subcore_barrier()
    pltpu.sync_copy(deg_sh.at[pl.ds(row0, ROWS_PER_TILE)],
                    out_hbm.at[cid, pl.ds(row0, ROWS_PER_TILE)])


def _make_sc_agg(d, stage_y):
    scratch = [
        pltpu.VMEM((NCH, ECH), jnp.int32),
        pltpu.VMEM((NCH, ECH), jnp.int32),
        pltpu.VMEM((ECH, d), jnp.float32),
        pltpu.VMEM_SHARED((N_PAD, d), jnp.float32),
        pltpu.SemaphoreType.DMA,
    ]
    if stage_y:
        scratch.insert(4, pltpu.VMEM_SHARED((N_PAD, d), jnp.float32))

    @functools.partial(
        pl.kernel,
        mesh=_mesh,
        out_type=jax.ShapeDtypeStruct((NC, N_PAD, d), jnp.float32),
        scratch_types=scratch,
        compiler_params=_sc_params,
    )
    def agg(y_hbm, src_hbm, dst_hbm, zeros_hbm, out_hbm,
            src_v, dst_v, rows_v, acc_sh, *rest):
        if stage_y:
            y_sh, sem = rest
        else:
            (sem,) = rest
        cid = lax.axis_index("c")
        sid = lax.axis_index("s")
        wid = sid * NC + cid
        row0 = sid * ROWS_PER_TILE
        pltpu.sync_copy(zeros_hbm, acc_sh.at[pl.ds(row0, ROWS_PER_TILE)])
        if stage_y:
            pltpu.sync_copy(y_hbm.at[pl.ds(row0, ROWS_PER_TILE)],
                            y_sh.at[pl.ds(row0, ROWS_PER_TILE)])
            y_src = y_sh
        else:
            y_src = y_hbm
        for j in range(NCH):
            pltpu.sync_copy(src_hbm.at[pl.ds(wid * EPT + j * ECH, ECH)], src_v.at[j])
            pltpu.sync_copy(dst_hbm.at[pl.ds(wid * EPT + j * ECH, ECH)], dst_v.at[j])
        plsc.subcore_barrier()

        def step(j, c):
            pltpu.async_copy(y_src.at[src_v.at[j]], rows_v, sem).wait()
            pltpu.sync_copy(rows_v, acc_sh.at[dst_v.at[j]], add=True)
            return c

        lax.fori_loop(0, NCH, step, 0)
        plsc.subcore_barrier()
        pltpu.sync_copy(acc_sh.at[pl.ds(row0, ROWS_PER_TILE)],
                        out_hbm.at[cid, pl.ds(row0, ROWS_PER_TILE)])

    return agg


_sc_agg64 = _make_sc_agg(64, stage_y=False)
_sc_agg32 = _make_sc_agg(32, stage_y=True)


_RB = N_PAD // 4
_NB = N_PAD // _RB


def _row_mask(i):
    return (i * _RB + lax.broadcasted_iota(jnp.int32, (_RB, 1), 0)) < N


def _tc_pre_body(deg_ref, x_ref, w1_ref, y1_ref, dinv_ref):
    i = pl.program_id(0)
    deg = deg_ref[0, :, 0:1] + deg_ref[1, :, 0:1] + 1.0
    dinv = lax.rsqrt(deg)
    xw = jnp.dot(x_ref[...], w1_ref[...], preferred_element_type=jnp.float32)
    y1_ref[...] = jnp.where(_row_mask(i), xw * dinv, 0.0)
    dinv_ref[...] = dinv


_tc_pre = pl.pallas_call(
    _tc_pre_body,
    grid=(_NB,),
    in_specs=[
        pl.BlockSpec((2, _RB, L), lambda i: (0, i, 0)),
        pl.BlockSpec((_RB, 128), lambda i: (i, 0)),
        pl.BlockSpec((128, 64), lambda i: (0, 0)),
    ],
    out_specs=(
        pl.BlockSpec((_RB, 64), lambda i: (i, 0)),
        pl.BlockSpec((_RB, 1), lambda i: (i, 0)),
    ),
    out_shape=(
        jax.ShapeDtypeStruct((N_PAD, 64), jnp.float32),
        jax.ShapeDtypeStruct((N_PAD, 1), jnp.float32),
    ),
)


def _tc_mid_body(acc_ref, y1_ref, dinv_ref, b1_ref, w2_ref, y2_ref):
    i = pl.program_id(0)
    dinv = dinv_ref[...]
    h1 = jnp.maximum(dinv * (acc_ref[0] + acc_ref[1] + y1_ref[...]) + b1_ref[...], 0.0)
    xw2 = jnp.dot(h1, w2_ref[...], preferred_element_type=jnp.float32)
    y2_ref[...] = jnp.where(_row_mask(i), xw2 * dinv, 0.0)


_tc_mid = pl.pallas_call(
    _tc_mid_body,
    grid=(_NB,),
    in_specs=[
        pl.BlockSpec((2, _RB, 64), lambda i: (0, i, 0)),
        pl.BlockSpec((_RB, 64), lambda i: (i, 0)),
        pl.BlockSpec((_RB, 1), lambda i: (i, 0)),
        pl.BlockSpec((1, 64), lambda i: (0, 0)),
        pl.BlockSpec((64, 32), lambda i: (0, 0)),
    ],
    out_specs=pl.BlockSpec((_RB, 32), lambda i: (i, 0)),
    out_shape=jax.ShapeDtypeStruct((N_PAD, 32), jnp.float32),
)


def _tc_post_body(acc_ref, y2_ref, dinv_ref, b2_ref, wlin_ref, blin_ref, out_ref):
    dinv = dinv_ref[...]
    h2 = jnp.maximum(dinv * (acc_ref[0] + acc_ref[1] + y2_ref[...]) + b2_ref[...], 0.0)
    out_ref[...] = (
        jnp.dot(h2, wlin_ref[...], preferred_element_type=jnp.float32) + blin_ref[...]
    )


_tc_post = pl.pallas_call(
    _tc_post_body,
    grid=(_NB,),
    in_specs=[
        pl.BlockSpec((2, _RB, 32), lambda i: (0, i, 0)),
        pl.BlockSpec((_RB, 32), lambda i: (i, 0)),
        pl.BlockSpec((_RB, 1), lambda i: (i, 0)),
        pl.BlockSpec((1, 32), lambda i: (0, 0)),
        pl.BlockSpec((32, 1), lambda i: (0, 0)),
        pl.BlockSpec((1, 1), lambda i: (0, 0)),
    ],
    out_specs=pl.BlockSpec((_RB, 1), lambda i: (i, 0)),
    out_shape=jax.ShapeDtypeStruct((N_PAD, 1), jnp.float32),
)


@jax.jit
def _run(x, edge_index, W1, b1, W2, b2, Wlin, blin):
    ei = jnp.asarray(edge_index, jnp.int32)
    src_r = ei[0]
    dst_r = ei[1]

    zeros64 = jnp.zeros((ROWS_PER_TILE, 64), jnp.float32)
    zeros32 = jnp.zeros((ROWS_PER_TILE, 32), jnp.float32)
    zeros16 = jnp.zeros((ROWS_PER_TILE, L), jnp.float32)
    ones16 = jnp.ones((ECH, L), jnp.float32)

    deg_parts = _sc_deg(dst_r, ones16, zeros16)
    y1, dinv = _tc_pre(deg_parts, x, W1)
    acc1 = _sc_agg64(y1, src_r, dst_r, zeros64)
    y2 = _tc_mid(acc1, y1, dinv, b1.reshape(1, 64), W2)
    acc2 = _sc_agg32(y2, src_r, dst_r, zeros32)
    out = _tc_post(acc2, y2, dinv, b2.reshape(1, 32), Wlin, blin.reshape(1, 1))
    return out[:N]


def kernel(x, edge_index, W1, b1, W2, b2, Wlin, blin):
    return _run(x, edge_index, W1, b1, W2, b2, Wlin, blin)


# speedup vs baseline: 1.1051x; 1.1051x over previous
import functools

import jax
import jax.numpy as jnp
from jax import lax
from jax.experimental import pallas as pl
from jax.experimental.pallas import tpu as pltpu
from jax.experimental.pallas import tpu_sc as plsc

NC = 2
NS = 16
NW = NC * NS
L = 16

N = 10000
E = 320000
N_PAD = 10112
ROWS_PER_TILE = N_PAD // NS
ECH = 1000
EPT = E // NW
NCH = EPT // ECH

_mesh = plsc.VectorSubcoreMesh(core_axis_name="c", subcore_axis_name="s")
_sc_params = pltpu.CompilerParams(use_tc_tiling_on_sc=False)


@functools.partial(
    pl.kernel,
    mesh=_mesh,
    out_type=jax.ShapeDtypeStruct((NC, N_PAD, L), jnp.float32),
    scratch_types=[
        pltpu.VMEM((NCH, ECH), jnp.int32),
        pltpu.VMEM((ECH, L), jnp.float32),
        pltpu.VMEM_SHARED((N_PAD, L), jnp.float32),
    ],
    compiler_params=_sc_params,
)
def _sc_deg(dst_hbm, ones_hbm, zeros_hbm, out_hbm, dst_v, ones_v, deg_sh):
    cid = lax.axis_index("c")
    sid = lax.axis_index("s")
    wid = sid * NC + cid
    row0 = sid * ROWS_PER_TILE
    pltpu.sync_copy(zeros_hbm, deg_sh.at[pl.ds(row0, ROWS_PER_TILE)])
    pltpu.sync_copy(ones_hbm, ones_v)
    pltpu.sync_copy(dst_hbm.at[wid], dst_v)
    plsc.subcore_barrier()

    def step(j, c):
        pltpu.sync_copy(ones_v, deg_sh.at[dst_v.at[j]], add=True)
        return c

    lax.fori_loop(0, NCH, step, 0)
    plsc.subcore_barrier()
    pltpu.sync_copy(deg_sh.at[pl.ds(row0, ROWS_PER_TILE)],
                    out_hbm.at[cid, pl.ds(row0, ROWS_PER_TILE)])


def _make_sc_agg(d, stage_y):
    scratch = [
        pltpu.VMEM((NCH, ECH), jnp.int32),
        pltpu.VMEM((NCH, ECH), jnp.int32),
        pltpu.VMEM((ECH, d), jnp.float32),
        pltpu.VMEM_SHARED((N_PAD, d), jnp.float32),
        pltpu.SemaphoreType.DMA,
    ]
    if stage_y:
        scratch.insert(4, pltpu.VMEM_SHARED((N_PAD, d), jnp.float32))

    @functools.partial(
        pl.kernel,
        mesh=_mesh,
        out_type=jax.ShapeDtypeStruct((NC, N_PAD, d), jnp.float32),
        scratch_types=scratch,
        compiler_params=_sc_params,
    )
    def agg(y_hbm, src_hbm, dst_hbm, zeros_hbm, out_hbm,
            src_v, dst_v, rows_v, acc_sh, *rest):
        if stage_y:
            y_sh, sem = rest
        else:
            (sem,) = rest
        cid = lax.axis_index("c")
        sid = lax.axis_index("s")
        wid = sid * NC + cid
        row0 = sid * ROWS_PER_TILE
        pltpu.sync_copy(zeros_hbm, acc_sh.at[pl.ds(row0, ROWS_PER_TILE)])
        if stage_y:
            pltpu.sync_copy(y_hbm.at[pl.ds(row0, ROWS_PER_TILE)],
                            y_sh.at[pl.ds(row0, ROWS_PER_TILE)])
            y_src = y_sh
        else:
            y_src = y_hbm
        pltpu.sync_copy(src_hbm.at[wid], src_v)
        pltpu.sync_copy(dst_hbm.at[wid], dst_v)
        plsc.subcore_barrier()

        def step(j, c):
            pltpu.async_copy(y_src.at[src_v.at[j]], rows_v, sem).wait()
            pltpu.sync_copy(rows_v, acc_sh.at[dst_v.at[j]], add=True)
            return c

        lax.fori_loop(0, NCH, step, 0)
        plsc.subcore_barrier()
        pltpu.sync_copy(acc_sh.at[pl.ds(row0, ROWS_PER_TILE)],
                        out_hbm.at[cid, pl.ds(row0, ROWS_PER_TILE)])

    return agg


_sc_agg64 = _make_sc_agg(64, stage_y=False)
_sc_agg32 = _make_sc_agg(32, stage_y=True)


_RB = N_PAD // 4
_NB = N_PAD // _RB


def _row_mask(i):
    return (i * _RB + lax.broadcasted_iota(jnp.int32, (_RB, 1), 0)) < N


def _tc_pre_body(deg_ref, x_ref, w1_ref, y1_ref, dinv_ref):
    i = pl.program_id(0)
    deg = deg_ref[0, :, 0:1] + deg_ref[1, :, 0:1] + 1.0
    dinv = lax.rsqrt(deg)
    xw = jnp.dot(x_ref[...], w1_ref[...], preferred_element_type=jnp.float32)
    y1_ref[...] = jnp.where(_row_mask(i), xw * dinv, 0.0)
    dinv_ref[...] = dinv


_tc_pre = pl.pallas_call(
    _tc_pre_body,
    grid=(_NB,),
    in_specs=[
        pl.BlockSpec((2, _RB, L), lambda i: (0, i, 0)),
        pl.BlockSpec((_RB, 128), lambda i: (i, 0)),
        pl.BlockSpec((128, 64), lambda i: (0, 0)),
    ],
    out_specs=(
        pl.BlockSpec((_RB, 64), lambda i: (i, 0)),
        pl.BlockSpec((_RB, 1), lambda i: (i, 0)),
    ),
    out_shape=(
        jax.ShapeDtypeStruct((N_PAD, 64), jnp.float32),
        jax.ShapeDtypeStruct((N_PAD, 1), jnp.float32),
    ),
)


def _tc_mid_body(acc_ref, y1_ref, dinv_ref, b1_ref, w2_ref, y2_ref):
    i = pl.program_id(0)
    dinv = dinv_ref[...]
    h1 = jnp.maximum(dinv * (acc_ref[0] + acc_ref[1] + y1_ref[...]) + b1_ref[...], 0.0)
    xw2 = jnp.dot(h1, w2_ref[...], preferred_element_type=jnp.float32)
    y2_ref[...] = jnp.where(_row_mask(i), xw2 * dinv, 0.0)


_tc_mid = pl.pallas_call(
    _tc_mid_body,
    grid=(_NB,),
    in_specs=[
        pl.BlockSpec((2, _RB, 64), lambda i: (0, i, 0)),
        pl.BlockSpec((_RB, 64), lambda i: (i, 0)),
        pl.BlockSpec((_RB, 1), lambda i: (i, 0)),
        pl.BlockSpec((1, 64), lambda i: (0, 0)),
        pl.BlockSpec((64, 32), lambda i: (0, 0)),
    ],
    out_specs=pl.BlockSpec((_RB, 32), lambda i: (i, 0)),
    out_shape=jax.ShapeDtypeStruct((N_PAD, 32), jnp.float32),
)


def _tc_post_body(acc_ref, y2_ref, dinv_ref, b2_ref, wlin_ref, blin_ref, out_ref):
    dinv = dinv_ref[...]
    h2 = jnp.maximum(dinv * (acc_ref[0] + acc_ref[1] + y2_ref[...]) + b2_ref[...], 0.0)
    out_ref[...] = (
        jnp.dot(h2, wlin_ref[...], preferred_element_type=jnp.float32) + blin_ref[...]
    )


_tc_post = pl.pallas_call(
    _tc_post_body,
    grid=(_NB,),
    in_specs=[
        pl.BlockSpec((2, _RB, 32), lambda i: (0, i, 0)),
        pl.BlockSpec((_RB, 32), lambda i: (i, 0)),
        pl.BlockSpec((_RB, 1), lambda i: (i, 0)),
        pl.BlockSpec((1, 32), lambda i: (0, 0)),
        pl.BlockSpec((32, 1), lambda i: (0, 0)),
        pl.BlockSpec((1, 1), lambda i: (0, 0)),
    ],
    out_specs=pl.BlockSpec((_RB, 1), lambda i: (i, 0)),
    out_shape=jax.ShapeDtypeStruct((N, 1), jnp.float32),
)


@jax.jit
def _run(x, edge_index, W1, b1, W2, b2, Wlin, blin):
    ei = jnp.asarray(edge_index, jnp.int32)
    src_r = ei[0].reshape(NW, NCH, ECH)
    dst_r = ei[1].reshape(NW, NCH, ECH)

    zeros64 = jnp.zeros((ROWS_PER_TILE, 64), jnp.float32)
    zeros32 = jnp.zeros((ROWS_PER_TILE, 32), jnp.float32)
    zeros16 = jnp.zeros((ROWS_PER_TILE, L), jnp.float32)
    ones16 = jnp.ones((ECH, L), jnp.float32)

    deg_parts = _sc_deg(dst_r, ones16, zeros16)
    y1, dinv = _tc_pre(deg_parts, x, W1)
    acc1 = _sc_agg64(y1, src_r, dst_r, zeros64)
    y2 = _tc_mid(acc1, y1, dinv, b1.reshape(1, 64), W2)
    acc2 = _sc_agg32(y2, src_r, dst_r, zeros32)
    out = _tc_post(acc2, y2, dinv, b2.reshape(1, 32), Wlin, blin.reshape(1, 1))
    return out


def kernel(x, edge_index, W1, b1, W2, b2, Wlin, blin):
    return _run(x, edge_index, W1, b1, W2, b2, Wlin, blin)
